# Initial kernel scaffold; baseline (speedup 1.0000x reference)
import jax
import jax.numpy as jnp
from jax.experimental import pallas as pl


def kernel(x, weight):
    raise NotImplementedError("write your pallas kernel here")


# speedup vs baseline: 1.5747x; 1.5747x over previous
import functools

import jax
import jax.numpy as jnp
from jax import lax
from jax.experimental import pallas as pl
from jax.experimental.pallas import tpu as pltpu
from jax.experimental.pallas import tpu_sc as plsc

VOCAB = 1_000_000
EMBED_DIM = 64
BATCH = 16384
HIST = 50
B_TOT = BATCH * HIST

_NC = 2
_NS = 16
NW = _NC * _NS
B_PER_W = B_TOT // NW
CHUNK = 128
NCHUNK = B_PER_W // CHUNK


def _body(x_hbm, w_hbm, out_hbm, idx_v, rows_v, sem):
    wid = lax.axis_index("s") * _NC + lax.axis_index("c")
    base = wid * B_PER_W

    def chunk_body(i, carry):
        off = base + i * CHUNK
        pltpu.sync_copy(x_hbm.at[pl.ds(off, CHUNK)], idx_v)
        pltpu.async_copy(w_hbm.at[idx_v], rows_v, sem).wait()
        pltpu.sync_copy(rows_v, out_hbm.at[pl.ds(off, CHUNK)])
        return carry

    lax.fori_loop(0, NCHUNK, chunk_body, 0)


def kernel(x, weight):
    xf = x.reshape(-1)
    mesh = plsc.VectorSubcoreMesh(core_axis_name="c", subcore_axis_name="s")
    run = functools.partial(
        pl.kernel,
        mesh=mesh,
        out_type=jax.ShapeDtypeStruct((B_TOT, EMBED_DIM), jnp.float32),
        scratch_types=[
            pltpu.VMEM((CHUNK,), jnp.int32),
            pltpu.VMEM((CHUNK, EMBED_DIM), jnp.float32),
            pltpu.SemaphoreType.DMA,
        ],
        compiler_params=pltpu.CompilerParams(use_tc_tiling_on_sc=False),
    )(_body)
    out = run(xf, weight)
    return out.reshape(BATCH, HIST, EMBED_DIM)


# speedup vs baseline: 1.8730x; 1.1894x over previous
import functools

import jax
import jax.numpy as jnp
from jax import lax
from jax.experimental import pallas as pl
from jax.experimental.pallas import tpu as pltpu
from jax.experimental.pallas import tpu_sc as plsc

VOCAB = 1_000_000
EMBED_DIM = 64
BATCH = 16384
HIST = 50
B_TOT = BATCH * HIST

_NC = 2
_NS = 16
NW = _NC * _NS
B_PER_W = B_TOT // NW

NBUF = 8
CHUNK = 160
NCHUNK = B_PER_W // CHUNK
NGROUP = NCHUNK // NBUF


def _body(x_hbm, w_hbm, out_hbm, idx_v, rows_v, sem_g, sem_w):
    wid = lax.axis_index("s") * _NC + lax.axis_index("c")
    base = wid * B_PER_W
    pltpu.sync_copy(x_hbm.at[pl.ds(base, B_PER_W)], idx_v)

    def idx_slice(t):
        return idx_v.at[pl.ds(t * CHUNK, CHUNK)]

    def out_slice(t):
        return out_hbm.at[pl.ds(base + t * CHUNK, CHUNK)]

    for b in range(NBUF):
        pltpu.async_copy(w_hbm.at[idx_slice(b)], rows_v.at[b], sem_g.at[b])

    def group(g, carry):
        t0 = g * NBUF
        for b in range(NBUF):
            t = t0 + b
            pltpu.make_async_copy(
                w_hbm.at[idx_slice(t)], rows_v.at[b], sem_g.at[b]
            ).wait()
            pltpu.async_copy(rows_v.at[b], out_slice(t), sem_w.at[b])
        for b in range(NBUF):
            t = t0 + b
            pltpu.make_async_copy(
                rows_v.at[b], out_slice(t), sem_w.at[b]
            ).wait()
            pltpu.async_copy(
                w_hbm.at[idx_slice(t + NBUF)], rows_v.at[b], sem_g.at[b]
            )
        return carry

    lax.fori_loop(0, NGROUP - 1, group, 0)

    t0 = (NGROUP - 1) * NBUF
    for b in range(NBUF):
        t = t0 + b
        pltpu.make_async_copy(
            w_hbm.at[idx_slice(t)], rows_v.at[b], sem_g.at[b]
        ).wait()
        pltpu.async_copy(rows_v.at[b], out_slice(t), sem_w.at[b])
    for b in range(NBUF):
        t = t0 + b
        pltpu.make_async_copy(rows_v.at[b], out_slice(t), sem_w.at[b]).wait()


def kernel(x, weight):
    xf = x.reshape(-1)
    mesh = plsc.VectorSubcoreMesh(core_axis_name="c", subcore_axis_name="s")
    run = functools.partial(
        pl.kernel,
        mesh=mesh,
        out_type=jax.ShapeDtypeStruct((B_TOT, EMBED_DIM), jnp.float32),
        scratch_types=[
            pltpu.VMEM((B_PER_W,), jnp.int32),
            pltpu.VMEM((NBUF, CHUNK, EMBED_DIM), jnp.float32),
            pltpu.SemaphoreType.DMA((NBUF,)),
            pltpu.SemaphoreType.DMA((NBUF,)),
        ],
        compiler_params=pltpu.CompilerParams(use_tc_tiling_on_sc=False),
    )(_body)
    out = run(xf, weight)
    return out.reshape(BATCH, HIST, EMBED_DIM)
